# baseline (speedup 1.0000x reference)
import jax
import jax.numpy as jnp
from jax.experimental import pallas as pl
from jax.experimental.pallas import tpu as pltpu

DIM = 768
HID = 768
E = 8
T = 2048
CT = 512
EPG = 2
HH = HID // 2


def _moe_body(x_ref, rW_ref, rb_ref, W1_ref, b1_ref, W2_ref, b2_ref,
              out_ref, w_scratch, acc_ref):
    g = pl.program_id(0)
    s1 = (g % 2) == 1

    @pl.when(g == 0)
    def _router():
        logits = jnp.dot(x_ref[...], rW_ref[...],
                         preferred_element_type=jnp.float32)
        logits = logits + rb_ref[0]
        m = jnp.max(logits, axis=-1, keepdims=True)
        p = jnp.exp(logits - m)
        w_scratch[...] = p / jnp.sum(p, axis=-1, keepdims=True)

    first_pair = g == 1
    for c in range(T // CT):
        sl = pl.ds(c * CT, CT)
        xs = x_ref[sl, :]
        eos = []
        for j in range(EPG):
            h = jnp.dot(xs, W1_ref[j], preferred_element_type=jnp.float32)
            h = h + b1_ref[j, 0]
            h = 0.5 * h * (1.0 + jax.lax.erf(h * 0.7071067811865476))
            eos.append(jnp.dot(h, W2_ref[j], preferred_element_type=jnp.float32))

        @pl.when(~s1)
        def _stash(eos=eos, sl=sl):
            for j in range(EPG):
                acc_ref[j, sl, :] = eos[j]

        @pl.when(s1)
        def _combine(eos=eos, sl=sl):
            lane = jax.lax.broadcasted_iota(jnp.int32, (CT, E), 1)
            ws = w_scratch[sl, :]
            contrib = jnp.zeros((CT, DIM), jnp.float32)
            for j in range(EPG):
                e = (g // 2) * EPG + j
                eo = acc_ref[j, sl, :] + eos[j] + b2_ref[j, 0]
                w_e = jnp.sum(jnp.where(lane == e, ws, 0.0), axis=-1,
                              keepdims=True)
                contrib = contrib + w_e * eo
            out_ref[sl, :] = jnp.where(first_pair, contrib,
                                       out_ref[sl, :] + contrib)


def kernel(x, rW, rb, W1, b1, W2, b2):
    B, Tx, D = x.shape
    x2 = x.reshape(Tx, D)
    out = pl.pallas_call(
        _moe_body,
        grid=(2 * E // EPG,),
        in_specs=[
            pl.BlockSpec((T, DIM), lambda g: (0, 0)),
            pl.BlockSpec((DIM, E), lambda g: (0, 0)),
            pl.BlockSpec((1, E), lambda g: (0, 0)),
            pl.BlockSpec((EPG, DIM, HH), lambda g: (g // 2, 0, g % 2)),
            pl.BlockSpec((EPG, 1, HH), lambda g: (g // 2, 0, g % 2)),
            pl.BlockSpec((EPG, HH, DIM), lambda g: (g // 2, g % 2, 0)),
            pl.BlockSpec((EPG, 1, DIM), lambda g: (g // 2, 0, 0)),
        ],
        out_specs=pl.BlockSpec((T, DIM), lambda g: (0, 0)),
        out_shape=jax.ShapeDtypeStruct((Tx, DIM), jnp.float32),
        scratch_shapes=[pltpu.VMEM((T, E), jnp.float32),
                        pltpu.VMEM((EPG, T, DIM), jnp.float32)],
        compiler_params=pltpu.CompilerParams(
            dimension_semantics=("arbitrary",),
        ),
    )(x2, rW, rb.reshape(1, E), W1, b1.reshape(E, 1, HID), W2,
      b2.reshape(E, 1, DIM))
    return out.reshape(B, Tx, D)


# speedup vs baseline: 1.5459x; 1.5459x over previous
import jax
import jax.numpy as jnp
from jax.experimental import pallas as pl
from jax.experimental.pallas import tpu as pltpu

DIM = 768
HID = 768
E = 8
T = 2048
CT = 512
EPG = 2


def _moe_body(x_ref, rW_ref, rb_ref, W1_ref, b1_ref, W2_ref, b2_ref,
              out_ref, w_scratch):
    g = pl.program_id(0)

    @pl.when(g == 0)
    def _router():
        logits = jnp.dot(x_ref[...], rW_ref[...],
                         preferred_element_type=jnp.float32)
        logits = logits + rb_ref[0]
        m = jnp.max(logits, axis=-1, keepdims=True)
        p = jnp.exp(logits - m)
        w_scratch[...] = p / jnp.sum(p, axis=-1, keepdims=True)

    first = g == 0
    for c in range(T // CT):
        sl = pl.ds(c * CT, CT)
        xs = x_ref[sl, :]
        lane = jax.lax.broadcasted_iota(jnp.int32, (CT, E), 1)
        ws = w_scratch[sl, :]
        contrib = jnp.zeros((CT, DIM), jnp.float32)
        for j in range(EPG):
            e = g * EPG + j
            h = jnp.dot(xs, W1_ref[j], preferred_element_type=jnp.float32)
            h = h + b1_ref[j, 0]
            h = 0.5 * h * (1.0 + jax.lax.erf(h * 0.7071067811865476))
            eo = jnp.dot(h, W2_ref[j], preferred_element_type=jnp.float32)
            eo = eo + b2_ref[j, 0]
            w_e = jnp.sum(jnp.where(lane == e, ws, 0.0), axis=-1,
                          keepdims=True)
            contrib = contrib + w_e * eo
        out_ref[sl, :] = jnp.where(first, contrib, out_ref[sl, :] + contrib)


def kernel(x, rW, rb, W1, b1, W2, b2):
    B, Tx, D = x.shape
    x2 = x.reshape(Tx, D)
    out = pl.pallas_call(
        _moe_body,
        grid=(E // EPG,),
        in_specs=[
            pl.BlockSpec((T, DIM), lambda g: (0, 0)),
            pl.BlockSpec((DIM, E), lambda g: (0, 0)),
            pl.BlockSpec((1, E), lambda g: (0, 0)),
            pl.BlockSpec((EPG, DIM, HID), lambda g: (g, 0, 0)),
            pl.BlockSpec((EPG, 1, HID), lambda g: (g, 0, 0)),
            pl.BlockSpec((EPG, HID, DIM), lambda g: (g, 0, 0)),
            pl.BlockSpec((EPG, 1, DIM), lambda g: (g, 0, 0)),
        ],
        out_specs=pl.BlockSpec((T, DIM), lambda g: (0, 0)),
        out_shape=jax.ShapeDtypeStruct((Tx, DIM), jnp.float32),
        scratch_shapes=[pltpu.VMEM((T, E), jnp.float32)],
        compiler_params=pltpu.CompilerParams(
            dimension_semantics=("arbitrary",),
        ),
    )(x2, rW, rb.reshape(1, E), W1, b1.reshape(E, 1, HID), W2,
      b2.reshape(E, 1, DIM))
    return out.reshape(B, Tx, D)


# speedup vs baseline: 1.5666x; 1.0134x over previous
import jax
import jax.numpy as jnp
from jax.experimental import pallas as pl
from jax.experimental.pallas import tpu as pltpu

DIM = 768
HID = 768
E = 8
T = 2048
CT = 512
EPG = 1


def _moe_body(x_ref, rW_ref, rb_ref, W1_ref, b1_ref, W2_ref, b2_ref,
              out_ref, w_scratch):
    g = pl.program_id(0)

    @pl.when(g == 0)
    def _router():
        logits = jnp.dot(x_ref[...], rW_ref[...],
                         preferred_element_type=jnp.float32)
        logits = logits + rb_ref[0]
        m = jnp.max(logits, axis=-1, keepdims=True)
        p = jnp.exp(logits - m)
        w_scratch[...] = p / jnp.sum(p, axis=-1, keepdims=True)

    first = g == 0
    for c in range(T // CT):
        sl = pl.ds(c * CT, CT)
        xs = x_ref[sl, :]
        lane = jax.lax.broadcasted_iota(jnp.int32, (CT, E), 1)
        ws = w_scratch[sl, :]
        contrib = jnp.zeros((CT, DIM), jnp.float32)
        for j in range(EPG):
            e = g * EPG + j
            h = jnp.dot(xs, W1_ref[j], preferred_element_type=jnp.float32)
            h = h + b1_ref[j, 0]
            h = 0.5 * h * (1.0 + jax.lax.erf(h * 0.7071067811865476))
            eo = jnp.dot(h, W2_ref[j], preferred_element_type=jnp.float32)
            eo = eo + b2_ref[j, 0]
            w_e = jnp.sum(jnp.where(lane == e, ws, 0.0), axis=-1,
                          keepdims=True)
            contrib = contrib + w_e * eo
        out_ref[sl, :] = jnp.where(first, contrib, out_ref[sl, :] + contrib)


def kernel(x, rW, rb, W1, b1, W2, b2):
    B, Tx, D = x.shape
    x2 = x.reshape(Tx, D)
    out = pl.pallas_call(
        _moe_body,
        grid=(E // EPG,),
        in_specs=[
            pl.BlockSpec((T, DIM), lambda g: (0, 0)),
            pl.BlockSpec((DIM, E), lambda g: (0, 0)),
            pl.BlockSpec((1, E), lambda g: (0, 0)),
            pl.BlockSpec((EPG, DIM, HID), lambda g: (g, 0, 0)),
            pl.BlockSpec((EPG, 1, HID), lambda g: (g, 0, 0)),
            pl.BlockSpec((EPG, HID, DIM), lambda g: (g, 0, 0)),
            pl.BlockSpec((EPG, 1, DIM), lambda g: (g, 0, 0)),
        ],
        out_specs=pl.BlockSpec((T, DIM), lambda g: (0, 0)),
        out_shape=jax.ShapeDtypeStruct((Tx, DIM), jnp.float32),
        scratch_shapes=[pltpu.VMEM((T, E), jnp.float32)],
        compiler_params=pltpu.CompilerParams(
            dimension_semantics=("arbitrary",),
        ),
    )(x2, rW, rb.reshape(1, E), W1, b1.reshape(E, 1, HID), W2,
      b2.reshape(E, 1, DIM))
    return out.reshape(B, Tx, D)
